# baseline (speedup 1.0000x reference)
import jax
import jax.numpy as jnp
from jax.experimental import pallas as pl
from jax.experimental.pallas import tpu as pltpu

_STEPS = 32
_H = 224
_W = 224
_PACK = 4


def _ecc_kernel(ts_ref, sel_ref, x_ref, o_ref):
    x = x_ref[...]
    h, w = x.shape
    inf = jnp.float32(jnp.inf)
    inf_row = jnp.full((1, w), inf, jnp.float32)
    col = jax.lax.broadcasted_iota(jnp.int32, (1, w), 1)
    bias = jnp.where(col % _W == _W - 1, inf, jnp.float32(0.0))
    xr = jnp.concatenate([x[:, 1:], x[:, :1]], axis=1)
    hmax = jnp.maximum(x, xr) + bias
    xd = jnp.concatenate([x[1:, :], inf_row], axis=0)
    vmax = jnp.maximum(x, xd)
    hmax_d = jnp.concatenate([hmax[1:, :], inf_row], axis=0)
    smax = jnp.maximum(hmax, hmax_d)

    one = jnp.float32(1.0)
    zero = jnp.float32(0.0)
    parts = []
    for k in range(_STEPS):
        t = ts_ref[0, k]
        pos = jnp.where(x <= t, one, zero) + jnp.where(smax <= t, one, zero)
        neg = jnp.where(hmax <= t, one, zero) + jnp.where(vmax <= t, one, zero)
        parts.append(jnp.sum(pos - neg, axis=0, keepdims=True))
    s = jnp.concatenate(parts, axis=0)
    o_ref[0] = jnp.dot(s, sel_ref[...], preferred_element_type=jnp.float32)


def _ecc(x, *, interpret=False):
    b, c, h, w = x.shape
    n = b * c
    grid = n // _PACK
    xt = x.reshape(n, h, w).transpose(1, 0, 2).reshape(h, n * w)
    ts = jnp.linspace(0.0, 1.0, _STEPS).astype(jnp.float32).reshape(1, _STEPS)
    seg = jnp.arange(_PACK * w, dtype=jnp.int32) // w
    sel = (seg[:, None] == jnp.arange(_PACK, dtype=jnp.int32)[None, :])
    sel = sel.astype(jnp.float32)
    out = pl.pallas_call(
        _ecc_kernel,
        grid=(grid,),
        in_specs=[
            pl.BlockSpec(memory_space=pltpu.SMEM),
            pl.BlockSpec((_PACK * w, _PACK), lambda i: (0, 0)),
            pl.BlockSpec((h, _PACK * w), lambda i: (0, i)),
        ],
        out_specs=pl.BlockSpec((1, _STEPS, _PACK), lambda i: (i, 0, 0)),
        out_shape=jax.ShapeDtypeStruct((grid, _STEPS, _PACK), jnp.float32),
        compiler_params=pltpu.CompilerParams(
            dimension_semantics=("parallel",),
        ),
        name="cub_ecc",
        interpret=interpret,
    )(ts, sel, xt)
    return out.transpose(0, 2, 1).reshape(b, c * _STEPS)


def kernel(x):
    return _ecc(x)


# speedup vs baseline: 1.1411x; 1.1411x over previous
import jax
import jax.numpy as jnp
from jax.experimental import pallas as pl
from jax.experimental.pallas import tpu as pltpu

_STEPS = 32
_H = 224
_W = 224
_LANES = 128
_ROWS = (_H * _W) // _LANES
_PACK = 4


def _ecc_kernel(ts_ref, br_ref, bb_ref, x_ref, o_ref):
    x = x_ref[0]
    bias_r = br_ref[...]
    bias_b = bb_ref[...]
    up1 = jnp.concatenate([x[1:, :], x[:1, :]], axis=0)
    s1 = jnp.concatenate([x[:, 1:], up1[:, :1]], axis=1)
    up2 = jnp.concatenate([x[2:, :], x[:2, :]], axis=0)
    s224 = jnp.concatenate([up1[:, 96:], up2[:, :96]], axis=1)
    hmax = jnp.maximum(x, s1) + bias_r
    vmax = jnp.maximum(x, s224) + bias_b
    h_up1 = jnp.concatenate([hmax[1:, :], hmax[:1, :]], axis=0)
    h_up2 = jnp.concatenate([hmax[2:, :], hmax[:2, :]], axis=0)
    h224 = jnp.concatenate([h_up1[:, 96:], h_up2[:, :96]], axis=1)
    smax = jnp.maximum(hmax, h224) + bias_b

    one = jnp.float32(1.0)
    zero = jnp.float32(0.0)
    outs = []
    for i in range(_PACK):
        r0, r1 = i * _ROWS, (i + 1) * _ROWS
        xi, hi = x[r0:r1], hmax[r0:r1]
        vi, si = vmax[r0:r1], smax[r0:r1]
        parts = []
        for k in range(_STEPS):
            t = ts_ref[0, k]
            pos = jnp.where(xi <= t, one, zero) + jnp.where(si <= t, one, zero)
            neg = jnp.where(hi <= t, one, zero) + jnp.where(vi <= t, one, zero)
            parts.append(jnp.sum(pos - neg, axis=0, keepdims=True))
        s = jnp.concatenate(parts, axis=0)
        outs.append(jnp.sum(s, axis=1, keepdims=True))
    o_ref[0] = jnp.stack(outs, axis=0)


def _ecc(x, *, interpret=False):
    b, c, h, w = x.shape
    n = b * c
    grid = n // _PACK
    xs = x.reshape(grid, _PACK * _ROWS, _LANES)
    ts = jnp.linspace(0.0, 1.0, _STEPS).astype(jnp.float32).reshape(1, _STEPS)
    rr = jax.lax.broadcasted_iota(jnp.int32, (_PACK * _ROWS, _LANES), 0)
    ll = jax.lax.broadcasted_iota(jnp.int32, (_PACK * _ROWS, _LANES), 1)
    p = (rr % _ROWS) * _LANES + ll
    inf = jnp.float32(jnp.inf)
    zero = jnp.float32(0.0)
    bias_r = jnp.where(p % _W == _W - 1, inf, zero)
    bias_b = jnp.where(p >= (_H - 1) * _W, inf, zero)
    out = pl.pallas_call(
        _ecc_kernel,
        grid=(grid,),
        in_specs=[
            pl.BlockSpec(memory_space=pltpu.SMEM),
            pl.BlockSpec((_PACK * _ROWS, _LANES), lambda i: (0, 0)),
            pl.BlockSpec((_PACK * _ROWS, _LANES), lambda i: (0, 0)),
            pl.BlockSpec((1, _PACK * _ROWS, _LANES), lambda i: (i, 0, 0)),
        ],
        out_specs=pl.BlockSpec((1, _PACK, _STEPS, 1), lambda i: (i, 0, 0, 0)),
        out_shape=jax.ShapeDtypeStruct((grid, _PACK, _STEPS, 1), jnp.float32),
        compiler_params=pltpu.CompilerParams(
            dimension_semantics=("parallel",),
        ),
        name="cub_ecc",
        interpret=interpret,
    )(ts, bias_r, bias_b, xs)
    return out.reshape(b, c * _STEPS)


def kernel(x):
    return _ecc(x)
